# Initial kernel scaffold; baseline (speedup 1.0000x reference)
import jax
import jax.numpy as jnp
from jax.experimental import pallas as pl


def kernel(x, edge_index, W1, b1, W2, b2):
    raise NotImplementedError("write your pallas kernel here")


# speedup vs baseline: 1.5247x; 1.5247x over previous
import functools

import jax
import jax.numpy as jnp
from jax import lax
from jax.experimental import pallas as pl
from jax.experimental.pallas import tpu as pltpu
from jax.experimental.pallas import tpu_sc as plsc

N_NODES = 10000
N_EDGES = 160000
D = 256
H = 512
L = 16
CH = H // L

NC = 2
NS = 16
NW = NC * NS
EPW = N_EDGES // NW
B = 40
NB = EPW // B


def _tc_precompute(x, WaT, WbT, b1_row):
    RB = 1000

    def body(x_ref, wa_ref, wb_ref, b1_ref, p_ref, q_ref):
        xb = x_ref[...]
        p_ref[...] = (
            jnp.dot(xb, wa_ref[...], preferred_element_type=jnp.float32)
            + b1_ref[...]
        )
        q_ref[...] = jnp.dot(xb, wb_ref[...], preferred_element_type=jnp.float32)

    return pl.pallas_call(
        body,
        grid=(N_NODES // RB,),
        in_specs=[
            pl.BlockSpec((RB, D), lambda i: (i, 0)),
            pl.BlockSpec((D, H), lambda i: (0, 0)),
            pl.BlockSpec((D, H), lambda i: (0, 0)),
            pl.BlockSpec((1, H), lambda i: (0, 0)),
        ],
        out_specs=[
            pl.BlockSpec((RB, H), lambda i: (i, 0)),
            pl.BlockSpec((RB, H), lambda i: (i, 0)),
        ],
        out_shape=[
            jax.ShapeDtypeStruct((N_NODES, H), jnp.float32),
            jax.ShapeDtypeStruct((N_NODES, H), jnp.float32),
        ],
    )(x, WaT, WbT, b1_row)


def _sc_edge_stage(P, Q, src, dst, wvec):
    mesh = plsc.VectorSubcoreMesh(core_axis_name="c", subcore_axis_name="s")

    @functools.partial(
        pl.kernel,
        out_type=jax.ShapeDtypeStruct((N_EDGES, L), jnp.float32),
        mesh=mesh,
        scratch_types=[
            pltpu.VMEM((EPW,), jnp.int32),
            pltpu.VMEM((EPW,), jnp.int32),
            pltpu.VMEM((B, H), jnp.float32),
            pltpu.VMEM((B, H), jnp.float32),
            pltpu.VMEM((H + L,), jnp.float32),
            pltpu.VMEM((B, L), jnp.float32),
            pltpu.SemaphoreType.DMA,
            pltpu.SemaphoreType.DMA,
        ],
    )
    def k(p_hbm, q_hbm, src_hbm, dst_hbm, w_hbm, acc_hbm,
          sidx, didx, prow, qrow, wv, accbuf, sem_p, sem_q):
        c = lax.axis_index("c")
        s = lax.axis_index("s")
        wid = s * NC + c
        base = wid * EPW
        pltpu.sync_copy(w_hbm, wv)
        pltpu.sync_copy(src_hbm.at[pl.ds(base, EPW)], sidx)
        pltpu.sync_copy(dst_hbm.at[pl.ds(base, EPW)], didx)

        w2 = [wv[pl.ds(i * L, L)] for i in range(CH)]
        b2v = wv[pl.ds(H, L)]

        def block_body(j, _):
            off = j * B
            cp = pltpu.async_copy(p_hbm.at[sidx.at[pl.ds(off, B)]], prow, sem_p)
            cq = pltpu.async_copy(q_hbm.at[didx.at[pl.ds(off, B)]], qrow, sem_q)
            cp.wait()
            cq.wait()

            def edge_body(e, _):
                acc = b2v
                for i in range(CH):
                    pv = prow[e, pl.ds(i * L, L)]
                    qv = qrow[e, pl.ds(i * L, L)]
                    acc = acc + jnp.maximum(pv + qv, 0.0) * w2[i]
                accbuf[e] = acc
                return 0

            lax.fori_loop(0, B, edge_body, 0)
            pltpu.sync_copy(accbuf, acc_hbm.at[pl.ds(base + off, B)])
            return 0

        lax.fori_loop(0, NB, block_body, 0)

    return k(P, Q, src, dst, wvec)


def _tc_lane_sum(acc):
    RB = 20000

    def body(a_ref, o_ref):
        o_ref[...] = jnp.sum(a_ref[...], axis=1, keepdims=True)

    return pl.pallas_call(
        body,
        grid=(N_EDGES // RB,),
        in_specs=[pl.BlockSpec((RB, L), lambda i: (i, 0))],
        out_specs=pl.BlockSpec((RB, 1), lambda i: (i, 0)),
        out_shape=jax.ShapeDtypeStruct((N_EDGES, 1), jnp.float32),
    )(acc)


def kernel(x, edge_index, W1, b1, W2, b2):
    WaT = jnp.transpose(W1[:, :D])
    WbT = jnp.transpose(W1[:, D:])
    P, Q = _tc_precompute(x, WaT, WbT, b1.reshape(1, H))
    b2v = jnp.concatenate([b2.reshape(1), jnp.zeros((L - 1,), jnp.float32)])
    wvec = jnp.concatenate([W2.reshape(H), b2v])
    src = edge_index[0]
    dst = edge_index[1]
    acc = _sc_edge_stage(P, Q, src, dst, wvec)
    return _tc_lane_sum(acc)
